# Initial kernel scaffold; baseline (speedup 1.0000x reference)
import jax
import jax.numpy as jnp
from jax.experimental import pallas as pl


def kernel(pc1_l_s1, pc1_l_s4, feats, sa1_w1, sa1_b1, sa1_w2, sa1_b2, sa2_w1, sa2_b1, sa2_w2, sa2_b2, fc_w1, fc_b1, fc_w2, fc_b2):
    raise NotImplementedError("write your pallas kernel here")


# speedup vs baseline: 1.1971x; 1.1971x over previous
import functools

import jax
import jax.numpy as jnp
from jax.experimental import pallas as pl
from jax.experimental.pallas import tpu as pltpu

_B = 4
_NPOINT = 8192
_S = 2048
_K = 32
_D = 128


_FPS_R = 8
_FPS_C = _S // _FPS_R


def _fps_body(xyz_ref, out_ref):
    xs = [xyz_ref[b, 0] for b in range(_B)]
    ys = [xyz_ref[b, 1] for b in range(_B)]
    zs = [xyz_ref[b, 2] for b in range(_B)]
    lane_i = (
        jax.lax.broadcasted_iota(jnp.int32, (_FPS_R, _FPS_C), 0) * _FPS_C
        + jax.lax.broadcasted_iota(jnp.int32, (_FPS_R, _FPS_C), 1)
    )
    lane_f = lane_i.astype(jnp.float32)

    def body(i, state):
        new_state = []
        for b in range(_B):
            distance, far_f, cent_f = state[3 * b], state[3 * b + 1], state[3 * b + 2]
            cent_f = jnp.where(lane_i == i, far_f, cent_f)
            oh = lane_f == far_f
            cx = jnp.sum(jnp.where(oh, xs[b], 0.0))
            cy = jnp.sum(jnp.where(oh, ys[b], 0.0))
            cz = jnp.sum(jnp.where(oh, zs[b], 0.0))
            dx = xs[b] - cx
            dy = ys[b] - cy
            dz = zs[b] - cz
            dist = (dx * dx + dy * dy) + dz * dz
            distance = jnp.minimum(distance, dist)
            m = jnp.max(distance)
            far_new = jnp.min(jnp.where(distance == m, lane_f, float(_S)))
            new_state += [distance, far_new, cent_f]
        return tuple(new_state)

    state = []
    for _ in range(_B):
        state += [
            jnp.full((_FPS_R, _FPS_C), 1e10, jnp.float32),
            jnp.float32(0.0),
            jnp.zeros((_FPS_R, _FPS_C), jnp.float32),
        ]
    state = jax.lax.fori_loop(0, _S, body, tuple(state))
    for b in range(_B):
        out_ref[b] = state[3 * b + 2].astype(jnp.int32)


def _fps(xyz):
    out = pl.pallas_call(
        _fps_body,
        out_shape=jax.ShapeDtypeStruct((_B, _FPS_R, _FPS_C), jnp.int32),
    )(xyz.reshape(_B, 3, _FPS_R, _FPS_C))
    return out.reshape(_B, _S)


def _square_distance(src, dst):
    return (
        jnp.sum(src**2, -1)[:, :, None]
        - 2.0 * jnp.einsum("bnc,bmc->bnm", src, dst)
        + jnp.sum(dst**2, -1)[:, None, :]
    )


def _index_points(points, idx):
    return jax.vmap(lambda p, i: p[i])(points, idx)


def _instance_norm(x, eps=1e-5):
    m = jnp.mean(x, axis=(2, 3), keepdims=True)
    v = jnp.mean((x - m) ** 2, axis=(2, 3), keepdims=True)
    return (x - m) / jnp.sqrt(v + eps)


def _sa(xyz_t, new_xyz, idx, pts_t, w1, b1, w2, b2):
    grouped_xyz = _index_points(xyz_t, idx)
    grouped_xyz_norm = grouped_xyz - new_xyz[:, :, None, :]
    grouped_pts = _index_points(pts_t, idx)
    new_points = jnp.concatenate([grouped_xyz_norm, grouped_pts], axis=-1)
    xx = jnp.transpose(new_points, (0, 3, 2, 1))
    xx = jax.nn.relu(
        _instance_norm(jnp.einsum("oc,bcsn->bosn", w1, xx) + b1[None, :, None, None])
    )
    xx = jax.nn.relu(
        _instance_norm(jnp.einsum("oc,bcsn->bosn", w2, xx) + b2[None, :, None, None])
    )
    return jnp.max(xx, axis=2)


def kernel(pc1_l_s1, pc1_l_s4, feats, sa1_w1, sa1_b1, sa1_w2, sa1_b2,
           sa2_w1, sa2_b1, sa2_w2, sa2_b2, fc_w1, fc_b1, fc_w2, fc_b2):
    xyz_t = jnp.transpose(pc1_l_s4, (0, 2, 1))

    fps_idx = _fps(pc1_l_s4)
    new_xyz = _index_points(xyz_t, fps_idx)
    sqrdists = _square_distance(new_xyz, xyz_t)
    _, idx = jax.lax.top_k(-sqrdists, _K)

    x = _sa(xyz_t, new_xyz, idx, jnp.transpose(feats, (0, 2, 1)),
            sa1_w1, sa1_b1, sa1_w2, sa1_b2)
    x = _sa(xyz_t, new_xyz, idx, jnp.transpose(x, (0, 2, 1)),
            sa2_w1, sa2_b1, sa2_w2, sa2_b2)

    x = jnp.transpose(x, (0, 2, 1))
    x = jax.nn.relu(x @ fc_w1.T + fc_b1)
    x = x @ fc_w2.T + fc_b2
    flow_lr = jnp.transpose(x, (0, 2, 1))

    x1 = jnp.transpose(pc1_l_s1, (0, 2, 1))
    p2 = jnp.transpose(flow_lr, (0, 2, 1))
    dists = _square_distance(x1, xyz_t)
    neg_d, idx3 = jax.lax.top_k(-dists, 3)
    d = -neg_d
    dist_recip = 1.0 / (d + 1e-8)
    norm = jnp.sum(dist_recip, axis=2, keepdims=True)
    weight = dist_recip / norm
    interp = jnp.sum(_index_points(p2, idx3) * weight[..., None], axis=2)
    flow = jnp.transpose(interp, (0, 2, 1))
    return flow, flow_lr


# speedup vs baseline: 2.1900x; 1.8294x over previous
import functools

import jax
import jax.numpy as jnp
from jax.experimental import pallas as pl
from jax.experimental.pallas import tpu as pltpu

_B = 4
_NPOINT = 8192
_S = 2048
_K = 32
_D = 128


_FPS_R = 8
_FPS_C = _S // _FPS_R


def _fps_body(xyz_ref, out_ref):
    xs = [xyz_ref[b, 0] for b in range(_B)]
    ys = [xyz_ref[b, 1] for b in range(_B)]
    zs = [xyz_ref[b, 2] for b in range(_B)]
    lane_i = (
        jax.lax.broadcasted_iota(jnp.int32, (_FPS_R, _FPS_C), 0) * _FPS_C
        + jax.lax.broadcasted_iota(jnp.int32, (_FPS_R, _FPS_C), 1)
    )
    lane_f = lane_i.astype(jnp.float32)

    def body(i, state):
        new_state = []
        for b in range(_B):
            distance, far_f, cent_f = state[3 * b], state[3 * b + 1], state[3 * b + 2]
            cent_f = jnp.where(lane_i == i, far_f, cent_f)
            oh = lane_f == far_f
            cx = jnp.sum(jnp.where(oh, xs[b], 0.0))
            cy = jnp.sum(jnp.where(oh, ys[b], 0.0))
            cz = jnp.sum(jnp.where(oh, zs[b], 0.0))
            dx = xs[b] - cx
            dy = ys[b] - cy
            dz = zs[b] - cz
            dist = (dx * dx + dy * dy) + dz * dz
            distance = jnp.minimum(distance, dist)
            m = jnp.max(distance)
            far_new = jnp.min(jnp.where(distance == m, lane_f, float(_S)))
            new_state += [distance, far_new, cent_f]
        return tuple(new_state)

    state = []
    for _ in range(_B):
        state += [
            jnp.full((_FPS_R, _FPS_C), 1e10, jnp.float32),
            jnp.float32(0.0),
            jnp.zeros((_FPS_R, _FPS_C), jnp.float32),
        ]
    state = jax.lax.fori_loop(0, _S, body, tuple(state))
    for b in range(_B):
        out_ref[b] = state[3 * b + 2].astype(jnp.int32)


def _fps(xyz):
    out = pl.pallas_call(
        _fps_body,
        out_shape=jax.ShapeDtypeStruct((_B, _FPS_R, _FPS_C), jnp.int32),
    )(xyz.reshape(_B, 3, _FPS_R, _FPS_C))
    return out.reshape(_B, _S)


def _fc_body(x_ref, w1_ref, b1_ref, w2_ref, b2_ref, out_ref):
    x = x_ref[0]
    h = jnp.maximum(jnp.dot(w1_ref[...], x) + b1_ref[...], 0.0)
    out_ref[0] = jnp.dot(w2_ref[...], h) + b2_ref[...]


def _fc_head(x, fc_w1, fc_b1, fc_w2, fc_b2):
    return pl.pallas_call(
        _fc_body,
        grid=(_B,),
        in_specs=[
            pl.BlockSpec((1, _D, _S), lambda b: (b, 0, 0)),
            pl.BlockSpec((_D, _D), lambda b: (0, 0)),
            pl.BlockSpec((_D, 1), lambda b: (0, 0)),
            pl.BlockSpec((3, _D), lambda b: (0, 0)),
            pl.BlockSpec((3, 1), lambda b: (0, 0)),
        ],
        out_specs=pl.BlockSpec((1, 3, _S), lambda b: (b, 0, 0)),
        out_shape=jax.ShapeDtypeStruct((_B, 3, _S), jnp.float32),
    )(x, fc_w1, fc_b1.reshape(_D, 1), fc_w2, fc_b2.reshape(3, 1))


_FP_Q = 512


def _fp_body(q_ref, r_ref, p2_ref, out_ref):
    qc = q_ref[0]
    rt = r_ref[0]
    q_sq = jnp.sum(qc * qc, axis=0, keepdims=True)
    r_sq = jnp.sum(rt * rt, axis=1, keepdims=True)
    dT = (q_sq - 2.0 * jnp.dot(rt, qc)) + r_sq
    linf = jax.lax.broadcasted_iota(jnp.int32, (_S, _FP_Q), 0).astype(jnp.float32)

    d = dT
    ms, iis = [], []
    for _ in range(3):
        m = jnp.min(d, axis=0, keepdims=True)
        i = jnp.min(jnp.where(d == m, linf, float(_S)), axis=0, keepdims=True)
        ms.append(m)
        iis.append(i)
        d = jnp.where((d == m) & (linf == i), jnp.float32(jnp.inf), d)

    rec = [1.0 / (m + 1e-8) for m in ms]
    norm = (rec[0] + rec[1]) + rec[2]
    wt = jnp.zeros((_S, _FP_Q), jnp.float32)
    for k in range(3):
        wt = jnp.where(linf == iis[k], rec[k] / norm, wt)
    out_ref[0] = jax.lax.dot(
        p2_ref[0], wt, precision=jax.lax.Precision.HIGHEST
    )


def _feature_propagation(pc1, pc4_t, flow_lr):
    return pl.pallas_call(
        _fp_body,
        grid=(_B, _NPOINT // _FP_Q),
        in_specs=[
            pl.BlockSpec((1, 3, _FP_Q), lambda b, i: (b, 0, i)),
            pl.BlockSpec((1, _S, 3), lambda b, i: (b, 0, 0)),
            pl.BlockSpec((1, 3, _S), lambda b, i: (b, 0, 0)),
        ],
        out_specs=pl.BlockSpec((1, 3, _FP_Q), lambda b, i: (b, 0, i)),
        out_shape=jax.ShapeDtypeStruct((_B, 3, _NPOINT), jnp.float32),
    )(pc1, pc4_t, flow_lr)


def _square_distance(src, dst):
    return (
        jnp.sum(src**2, -1)[:, :, None]
        - 2.0 * jnp.einsum("bnc,bmc->bnm", src, dst)
        + jnp.sum(dst**2, -1)[:, None, :]
    )


def _index_points(points, idx):
    return jax.vmap(lambda p, i: p[i])(points, idx)


def _instance_norm(x, eps=1e-5):
    m = jnp.mean(x, axis=(2, 3), keepdims=True)
    v = jnp.mean((x - m) ** 2, axis=(2, 3), keepdims=True)
    return (x - m) / jnp.sqrt(v + eps)


def _sa(xyz_t, new_xyz, idx, pts_t, w1, b1, w2, b2):
    grouped_xyz = _index_points(xyz_t, idx)
    grouped_xyz_norm = grouped_xyz - new_xyz[:, :, None, :]
    grouped_pts = _index_points(pts_t, idx)
    new_points = jnp.concatenate([grouped_xyz_norm, grouped_pts], axis=-1)
    xx = jnp.transpose(new_points, (0, 3, 2, 1))
    xx = jax.nn.relu(
        _instance_norm(jnp.einsum("oc,bcsn->bosn", w1, xx) + b1[None, :, None, None])
    )
    xx = jax.nn.relu(
        _instance_norm(jnp.einsum("oc,bcsn->bosn", w2, xx) + b2[None, :, None, None])
    )
    return jnp.max(xx, axis=2)


def kernel(pc1_l_s1, pc1_l_s4, feats, sa1_w1, sa1_b1, sa1_w2, sa1_b2,
           sa2_w1, sa2_b1, sa2_w2, sa2_b2, fc_w1, fc_b1, fc_w2, fc_b2):
    xyz_t = jnp.transpose(pc1_l_s4, (0, 2, 1))

    fps_idx = _fps(pc1_l_s4)
    new_xyz = _index_points(xyz_t, fps_idx)
    sqrdists = _square_distance(new_xyz, xyz_t)
    _, idx = jax.lax.top_k(-sqrdists, _K)

    x = _sa(xyz_t, new_xyz, idx, jnp.transpose(feats, (0, 2, 1)),
            sa1_w1, sa1_b1, sa1_w2, sa1_b2)
    x = _sa(xyz_t, new_xyz, idx, jnp.transpose(x, (0, 2, 1)),
            sa2_w1, sa2_b1, sa2_w2, sa2_b2)

    flow_lr = _fc_head(x, fc_w1, fc_b1, fc_w2, fc_b2)
    flow = _feature_propagation(pc1_l_s1, xyz_t, flow_lr)
    return flow, flow_lr
